# baseline (speedup 1.0000x reference)
import functools

import jax
import jax.numpy as jnp
import numpy as np
from jax import lax
from jax.experimental import pallas as pl
from jax.experimental.pallas import tpu as pltpu
from jax.experimental.pallas import tpu_sc as plsc

N = 10000
HID = 128
NH = 8
HD = 16
NTAB = N + 8
NPAD = 10016
ROWS_PER_TILE = NPAD // 16
NW = 32
K = 80
SROW = 160
BR = 400
TPB = 25
FBR = 512

_S_NP = np.zeros((HID, NH), np.float32)
for _h in range(NH):
    _S_NP[_h * HD:(_h + 1) * HD, _h] = 1.0


def _leaky(x):
    return jnp.where(x < 0.0, x * 0.2, x)


def _prep_body(x_ref, w_ref, b_ref, asrc_ref, adst_ref, s_ref,
               h_ref, ss_ref, sd_ref, m_ref):
    i = pl.program_id(0)
    h = jnp.dot(x_ref[...], w_ref[0], preferred_element_type=jnp.float32)
    h = h + b_ref[0]
    h_ref[...] = h
    smat = s_ref[...]
    ss = jnp.dot(h * asrc_ref[0], smat, preferred_element_type=jnp.float32)
    sd = jnp.dot(h * adst_ref[0], smat, preferred_element_type=jnp.float32)
    ss_ref[...] = ss
    sd_ref[...] = sd
    cur = jnp.max(ss, axis=0, keepdims=True)

    @pl.when(i % TPB == 0)
    def _():
        m_ref[0] = cur

    @pl.when(i % TPB != 0)
    def _():
        m_ref[0] = jnp.maximum(m_ref[0], cur)


def _prep(X, W, B, ASRC, ADST, SMAT):
    return pl.pallas_call(
        _prep_body,
        grid=(2 * TPB,),
        in_specs=[
            pl.BlockSpec((BR, HID), lambda i: (i, 0)),
            pl.BlockSpec((1, HID, HID), lambda i: (i // TPB, 0, 0)),
            pl.BlockSpec((1, 1, HID), lambda i: (i // TPB, 0, 0)),
            pl.BlockSpec((1, 1, HID), lambda i: (i // TPB, 0, 0)),
            pl.BlockSpec((1, 1, HID), lambda i: (i // TPB, 0, 0)),
            pl.BlockSpec((HID, NH), lambda i: (0, 0)),
        ],
        out_specs=[
            pl.BlockSpec((BR, HID), lambda i: (i, 0)),
            pl.BlockSpec((BR, NH), lambda i: (i, 0)),
            pl.BlockSpec((BR, NH), lambda i: (i, 0)),
            pl.BlockSpec((1, 1, NH), lambda i: (i // TPB, 0, 0)),
        ],
        out_shape=[
            jax.ShapeDtypeStruct((2 * N, HID), jnp.float32),
            jax.ShapeDtypeStruct((2 * N, NH), jnp.float32),
            jax.ShapeDtypeStruct((2 * N, NH), jnp.float32),
            jax.ShapeDtypeStruct((2, 1, NH), jnp.float32),
        ],
    )(X, W, B, ASRC, ADST, SMAT)


def _tables_body(ss_ref, sd_ref, m_ref, st2_ref, dt_ref):
    ss = ss_ref[...]
    sd = sd_ref[...]
    st2_ref[...] = jnp.concatenate([ss, ss], axis=1)
    c = _leaky(m_ref[0] + sd)
    dt_ref[...] = jnp.concatenate([sd, -c], axis=1)


def _tables(SS, SD, M):
    return pl.pallas_call(
        _tables_body,
        grid=(2 * TPB,),
        in_specs=[
            pl.BlockSpec((BR, NH), lambda i: (i, 0)),
            pl.BlockSpec((BR, NH), lambda i: (i, 0)),
            pl.BlockSpec((1, 1, NH), lambda i: (1 - i // TPB, 0, 0)),
        ],
        out_specs=[
            pl.BlockSpec((BR, 2 * NH), lambda i: (i, 0)),
            pl.BlockSpec((BR, 2 * NH), lambda i: (i, 0)),
        ],
        out_shape=[
            jax.ShapeDtypeStruct((2 * N, 2 * NH), jnp.float32),
            jax.ShapeDtypeStruct((2 * N, 2 * NH), jnp.float32),
        ],
    )(SS, SD, M)


def _prop_body(src_tab, dst_tab, src_ids, dst_ids, zeros, out,
               sidx0, didx0, rows0, drows0, obuf0, didx0b,
               sidx1, didx1, rows1, drows1, obuf1, didx1b,
               acc, gsem0, gsem1, ssem0, ssem1):
    cid = lax.axis_index("c")
    sid = lax.axis_index("s")
    wid = sid * 2 + cid
    epw = src_ids.shape[0] // NW
    nch = epw // K

    pltpu.sync_copy(zeros, acc.at[pl.ds(sid * ROWS_PER_TILE, ROWS_PER_TILE)])
    plsc.subcore_barrier()

    base0 = wid * epw
    bufs = ((sidx0, didx0, rows0, drows0, obuf0, didx0b, gsem0, ssem0),
            (sidx1, didx1, rows1, drows1, obuf1, didx1b, gsem1, ssem1))

    def fetch(ci, b):
        s_, d_, r_, dr_, ob_, d2_, gs, _ = bufs[b]
        base = base0 + ci * K
        pltpu.sync_copy(src_ids.at[pl.ds(base, K)], s_)
        pltpu.sync_copy(dst_ids.at[pl.ds(base, K)], d_)
        pltpu.async_copy(src_tab.at[s_.at[pl.ds(0, K // 2)]],
                         r_.at[pl.ds(0, K // 2)], gs)
        pltpu.async_copy(src_tab.at[s_.at[pl.ds(K // 2, K // 2)]],
                         r_.at[pl.ds(K // 2, K // 2)], gs)
        pltpu.async_copy(dst_tab.at[d_], dr_, gs)

    def wait_gather(b):
        s_, d_, r_, dr_, ob_, d2_, gs, _ = bufs[b]
        pltpu.make_async_copy(src_tab.at[s_.at[pl.ds(0, K // 2)]],
                              r_.at[pl.ds(0, K // 2)], gs).wait()
        pltpu.make_async_copy(src_tab.at[s_.at[pl.ds(K // 2, K // 2)]],
                              r_.at[pl.ds(K // 2, K // 2)], gs).wait()
        pltpu.make_async_copy(dst_tab.at[d_], dr_, gs).wait()

    def wait_scatter(b):
        s_, d_, r_, dr_, ob_, d2_, _, ss = bufs[b]
        pltpu.make_async_copy(ob_, acc.at[d2_], ss).wait()

    def compute(b):
        s_, d_, r_, dr_, ob_, d2_, _, _ = bufs[b]
        rot_idx = (lax.iota(jnp.int32, HD) + NH) & (HD - 1)

        @plsc.parallel_loop(0, K, 1, unroll=4)
        def edge(e):
            shi, slo = plsc.unpack(r_[e, pl.ds(HID, 2 * HD)],
                                   format=plsc.PackFormat.INTERLEAVED,
                                   preferred_element_type=jnp.float32)
            ssv = shi + slo
            dv = dr_[e, pl.ds(0, HD)]
            rot = plsc.load_gather(dr_, [jnp.full((HD,), e, jnp.int32), rot_idx])
            t = ssv + dv
            t = jnp.where(t < 0.0, t * 0.2, t)
            w2 = jnp.exp(t + rot)
            ob_[e, pl.ds(HID, HD)] = w2
            for j in range(4):
                fv = r_[e, pl.ds(32 * j, 32)]
                va, vb = plsc.unpack(fv, format=plsc.PackFormat.INTERLEAVED,
                                     preferred_element_type=jnp.float32)
                ob_[e, pl.ds(32 * j, HD)] = va * w2[2 * j]
                ob_[e, pl.ds(32 * j + HD, HD)] = vb * w2[2 * j + 1]

    fetch(0, 0)

    def pair(pi, carry):
        for b in (0, 1):
            ci = pi * 2 + b
            s_, d_, r_, dr_, ob_, d2_, _, ss = bufs[b]

            @pl.when(ci + 1 < nch)
            def _():
                fetch(ci + 1, 1 - b)

            wait_gather(b)

            @pl.when(ci >= 2)
            def _():
                wait_scatter(b)

            for q in range(K // HD):
                d2_[pl.ds(q * HD, HD)] = d_[pl.ds(q * HD, HD)]
            compute(b)
            pltpu.async_copy(ob_, acc.at[d2_], ss, add=True)
        return carry

    lax.fori_loop(0, nch // 2, pair, 0)
    wait_scatter(0)
    wait_scatter(1)
    plsc.subcore_barrier()
    pltpu.sync_copy(acc.at[pl.ds(sid * ROWS_PER_TILE, ROWS_PER_TILE)],
                    out.at[cid, pl.ds(sid * ROWS_PER_TILE, ROWS_PER_TILE)])


_prop_sc = functools.partial(
    pl.kernel,
    mesh=plsc.VectorSubcoreMesh(core_axis_name="c", subcore_axis_name="s"),
    compiler_params=pltpu.CompilerParams(use_tc_tiling_on_sc=False,
                                         needs_layout_passes=False),
    out_type=jax.ShapeDtypeStruct((2, NPAD, HID + 2 * NH), jnp.float32),
    scratch_types=[
        pltpu.VMEM((K,), jnp.int32),
        pltpu.VMEM((K,), jnp.int32),
        pltpu.VMEM((K, SROW), jnp.bfloat16),
        pltpu.VMEM((K, 2 * NH), jnp.float32),
        pltpu.VMEM((K, HID + 2 * NH), jnp.float32),
        pltpu.VMEM((K,), jnp.int32),
        pltpu.VMEM((K,), jnp.int32),
        pltpu.VMEM((K,), jnp.int32),
        pltpu.VMEM((K, SROW), jnp.bfloat16),
        pltpu.VMEM((K, 2 * NH), jnp.float32),
        pltpu.VMEM((K, HID + 2 * NH), jnp.float32),
        pltpu.VMEM((K,), jnp.int32),
        pltpu.VMEM_SHARED((NPAD, HID + 2 * NH), jnp.float32),
        pltpu.SemaphoreType.DMA,
        pltpu.SemaphoreType.DMA,
        pltpu.SemaphoreType.DMA,
        pltpu.SemaphoreType.DMA,
    ],
)(_prop_body)


def _finish_body(acc_ref, r_ref, out_ref):
    s = acc_ref[0] + acc_ref[1]
    num = s[:, 0:HID]
    den = jnp.maximum(s[:, HID:HID + NH], 1e-16)
    dexp = jnp.dot(den, r_ref[...], preferred_element_type=jnp.float32)
    out_ref[...] = jnp.maximum(num / dexp, 0.0)


def _finish(ACC, RMAT):
    return pl.pallas_call(
        _finish_body,
        grid=(pl.cdiv(NPAD, FBR),),
        in_specs=[
            pl.BlockSpec((2, FBR, HID + 2 * NH), lambda i: (0, i, 0)),
            pl.BlockSpec((NH, HID), lambda i: (0, 0)),
        ],
        out_specs=pl.BlockSpec((FBR, HID), lambda i: (i, 0)),
        out_shape=jax.ShapeDtypeStruct((NPAD, HID), jnp.float32),
    )(ACC, RMAT)


def _pad_tab(t):
    return jnp.pad(t, ((0, NTAB - N), (0, 0)))


def kernel(x_author, x_paper, params, edge_writes, edge_rev):
    E = edge_writes.shape[1]
    epw = ((E + NW * 2 * K - 1) // (NW * 2 * K)) * 2 * K
    epad = NW * epw - E
    fill = jnp.full((epad,), N, jnp.int32)
    s1 = jnp.concatenate([edge_writes[0], fill])
    d1 = jnp.concatenate([edge_writes[1], fill])
    s2 = jnp.concatenate([edge_rev[0], fill])
    d2 = jnp.concatenate([edge_rev[1], fill])

    smat = jnp.asarray(_S_NP)
    rmat = jnp.asarray(_S_NP.T)
    zeros = jnp.zeros((ROWS_PER_TILE, HID + 2 * NH), jnp.float32)

    xa, xp = x_author, x_paper
    for li in (1, 2):
        L = params['L%d' % li]
        X = jnp.concatenate([xa, xp], axis=0)
        W = jnp.stack([L['Wp_a'], L['Wp_p']])
        B = jnp.stack([L['bp_a'], L['bp_p']]).reshape(2, 1, HID)
        ASRC = jnp.stack([L['as_e1'].reshape(-1),
                          L['as_e2'].reshape(-1)]).reshape(2, 1, HID)
        ADST = jnp.stack([L['ad_e2'].reshape(-1),
                          L['ad_e1'].reshape(-1)]).reshape(2, 1, HID)
        H_, SS, SD, M = _prep(X, W, B, ASRC, ADST, smat)
        ST2, DT = _tables(SS, SD, M)
        feat = (H_.astype(jnp.bfloat16)
                .reshape(-1, 4, 2, HD).transpose(0, 1, 3, 2).reshape(-1, HID))
        shi = ST2.astype(jnp.bfloat16)
        slo = (ST2 - shi.astype(jnp.float32)).astype(jnp.bfloat16)
        sint = jnp.stack([shi, slo], axis=-1).reshape(-1, 2 * HD)
        ST = jnp.concatenate([feat, sint], axis=1)
        st_a, st_p = _pad_tab(ST[:N]), _pad_tab(ST[N:])
        dt_a, dt_p = _pad_tab(DT[:N]), _pad_tab(DT[N:])
        acc_p = _prop_sc(st_a, dt_p, s1, d1, zeros)
        acc_a = _prop_sc(st_p, dt_a, s2, d2, zeros)
        xa = _finish(acc_a, rmat)[:N]
        xp = _finish(acc_p, rmat)[:N]
    return xa, xp


# speedup vs baseline: 1.0998x; 1.0998x over previous
import functools

import jax
import jax.numpy as jnp
import numpy as np
from jax import lax
from jax.experimental import pallas as pl
from jax.experimental.pallas import tpu as pltpu
from jax.experimental.pallas import tpu_sc as plsc

N = 10000
HID = 128
NH = 8
HD = 16
NTAB = N + 8
NPAD = 10016
ROWS_PER_TILE = NPAD // 16
NW = 32
K = 80
SROW = 160
BR = 400
TPB = 25
FBR = 512

_S_NP = np.zeros((HID, NH), np.float32)
for _h in range(NH):
    _S_NP[_h * HD:(_h + 1) * HD, _h] = 1.0


def _leaky(x):
    return jnp.where(x < 0.0, x * 0.2, x)


def _prep_body(x_ref, w_ref, b_ref, asrc_ref, adst_ref, s_ref,
               h_ref, ss_ref, sd_ref, m_ref):
    i = pl.program_id(0)
    h = jnp.dot(x_ref[...], w_ref[0], preferred_element_type=jnp.float32)
    h = h + b_ref[0]
    h_ref[...] = h
    smat = s_ref[...]
    ss = jnp.dot(h * asrc_ref[0], smat, preferred_element_type=jnp.float32)
    sd = jnp.dot(h * adst_ref[0], smat, preferred_element_type=jnp.float32)
    ss_ref[...] = ss
    sd_ref[...] = sd
    cur = jnp.max(ss, axis=0, keepdims=True)

    @pl.when(i % TPB == 0)
    def _():
        m_ref[0] = cur

    @pl.when(i % TPB != 0)
    def _():
        m_ref[0] = jnp.maximum(m_ref[0], cur)


def _prep(X, W, B, ASRC, ADST, SMAT):
    return pl.pallas_call(
        _prep_body,
        grid=(2 * TPB,),
        in_specs=[
            pl.BlockSpec((BR, HID), lambda i: (i, 0)),
            pl.BlockSpec((1, HID, HID), lambda i: (i // TPB, 0, 0)),
            pl.BlockSpec((1, 1, HID), lambda i: (i // TPB, 0, 0)),
            pl.BlockSpec((1, 1, HID), lambda i: (i // TPB, 0, 0)),
            pl.BlockSpec((1, 1, HID), lambda i: (i // TPB, 0, 0)),
            pl.BlockSpec((HID, NH), lambda i: (0, 0)),
        ],
        out_specs=[
            pl.BlockSpec((BR, HID), lambda i: (i, 0)),
            pl.BlockSpec((BR, NH), lambda i: (i, 0)),
            pl.BlockSpec((BR, NH), lambda i: (i, 0)),
            pl.BlockSpec((1, 1, NH), lambda i: (i // TPB, 0, 0)),
        ],
        out_shape=[
            jax.ShapeDtypeStruct((2 * N, HID), jnp.float32),
            jax.ShapeDtypeStruct((2 * N, NH), jnp.float32),
            jax.ShapeDtypeStruct((2 * N, NH), jnp.float32),
            jax.ShapeDtypeStruct((2, 1, NH), jnp.float32),
        ],
    )(X, W, B, ASRC, ADST, SMAT)


def _tables_body(ss_ref, sd_ref, m_ref, st2_ref, dt_ref):
    ss = ss_ref[...]
    sd = sd_ref[...]
    st2_ref[...] = jnp.concatenate([ss, ss], axis=1)
    c = _leaky(m_ref[0] + sd)
    dt_ref[...] = jnp.concatenate([sd, -c], axis=1)


def _tables(SS, SD, M):
    return pl.pallas_call(
        _tables_body,
        grid=(2 * TPB,),
        in_specs=[
            pl.BlockSpec((BR, NH), lambda i: (i, 0)),
            pl.BlockSpec((BR, NH), lambda i: (i, 0)),
            pl.BlockSpec((1, 1, NH), lambda i: (1 - i // TPB, 0, 0)),
        ],
        out_specs=[
            pl.BlockSpec((BR, 2 * NH), lambda i: (i, 0)),
            pl.BlockSpec((BR, 2 * NH), lambda i: (i, 0)),
        ],
        out_shape=[
            jax.ShapeDtypeStruct((2 * N, 2 * NH), jnp.float32),
            jax.ShapeDtypeStruct((2 * N, 2 * NH), jnp.float32),
        ],
    )(SS, SD, M)


def _prop_body(src_tab, dst_tab, ids, zeros, out,
               iidx0, rows0, drows0, obuf0, didx0b,
               iidx1, rows1, drows1, obuf1, didx1b,
               acc, gsem0, gsem1, ssem0, ssem1):
    cid = lax.axis_index("c")
    sid = lax.axis_index("s")
    wid = sid * 2 + cid
    nch = ids.shape[0] // (NW * 2 * K)

    pltpu.sync_copy(zeros, acc.at[pl.ds(sid * ROWS_PER_TILE, ROWS_PER_TILE)])
    plsc.subcore_barrier()

    base0 = wid * nch * 2 * K
    bufs = ((iidx0, rows0, drows0, obuf0, didx0b, gsem0, ssem0),
            (iidx1, rows1, drows1, obuf1, didx1b, gsem1, ssem1))

    def fetch(ci, b):
        i_, r_, dr_, ob_, d2_, gs, _ = bufs[b]
        base = base0 + ci * 2 * K
        pltpu.sync_copy(ids.at[pl.ds(base, 2 * K)], i_)
        pltpu.async_copy(src_tab.at[i_.at[pl.ds(0, K)]], r_, gs)
        pltpu.async_copy(dst_tab.at[i_.at[pl.ds(K, K)]], dr_, gs)

    def wait_gather(b):
        i_, r_, dr_, ob_, d2_, gs, _ = bufs[b]
        pltpu.make_async_copy(src_tab.at[i_.at[pl.ds(0, K)]], r_, gs).wait()
        pltpu.make_async_copy(dst_tab.at[i_.at[pl.ds(K, K)]], dr_, gs).wait()

    def wait_scatter(b):
        i_, r_, dr_, ob_, d2_, _, ss = bufs[b]
        pltpu.make_async_copy(ob_, acc.at[d2_], ss).wait()

    def compute(b):
        i_, r_, dr_, ob_, d2_, _, _ = bufs[b]
        rot_idx = (lax.iota(jnp.int32, HD) + NH) & (HD - 1)

        @plsc.parallel_loop(0, K, 1, unroll=4)
        def edge(e):
            shi, slo = plsc.unpack(r_[e, pl.ds(HID, 2 * HD)],
                                   format=plsc.PackFormat.INTERLEAVED,
                                   preferred_element_type=jnp.float32)
            ssv = shi + slo
            dv = dr_[e, pl.ds(0, HD)]
            rot = plsc.load_gather(dr_, [jnp.full((HD,), e, jnp.int32), rot_idx])
            t = ssv + dv
            t = jnp.where(t < 0.0, t * 0.2, t)
            w2 = jnp.exp(t + rot)
            ob_[e, pl.ds(HID, HD)] = w2
            for j in range(4):
                fv = r_[e, pl.ds(32 * j, 32)]
                va, vb = plsc.unpack(fv, format=plsc.PackFormat.INTERLEAVED,
                                     preferred_element_type=jnp.float32)
                ob_[e, pl.ds(32 * j, HD)] = va * w2[2 * j]
                ob_[e, pl.ds(32 * j + HD, HD)] = vb * w2[2 * j + 1]

    fetch(0, 0)

    def pair(pi, carry):
        for b in (0, 1):
            ci = pi * 2 + b
            i_, r_, dr_, ob_, d2_, _, ss = bufs[b]

            @pl.when(ci + 1 < nch)
            def _():
                fetch(ci + 1, 1 - b)

            wait_gather(b)

            @pl.when(ci >= 2)
            def _():
                wait_scatter(b)

            for q in range(K // HD):
                d2_[pl.ds(q * HD, HD)] = i_[pl.ds(K + q * HD, HD)]
            compute(b)
            pltpu.async_copy(ob_, acc.at[d2_], ss, add=True)
        return carry

    lax.fori_loop(0, nch // 2, pair, 0)
    wait_scatter(0)
    wait_scatter(1)
    plsc.subcore_barrier()
    pltpu.sync_copy(acc.at[pl.ds(sid * ROWS_PER_TILE, ROWS_PER_TILE)],
                    out.at[cid, pl.ds(sid * ROWS_PER_TILE, ROWS_PER_TILE)])


_prop_sc = functools.partial(
    pl.kernel,
    mesh=plsc.VectorSubcoreMesh(core_axis_name="c", subcore_axis_name="s"),
    compiler_params=pltpu.CompilerParams(use_tc_tiling_on_sc=False,
                                         needs_layout_passes=False),
    out_type=jax.ShapeDtypeStruct((2, NPAD, HID + 2 * NH), jnp.float32),
    scratch_types=[
        pltpu.VMEM((2 * K,), jnp.int32),
        pltpu.VMEM((K, SROW), jnp.bfloat16),
        pltpu.VMEM((K, 2 * NH), jnp.float32),
        pltpu.VMEM((K, HID + 2 * NH), jnp.float32),
        pltpu.VMEM((K,), jnp.int32),
        pltpu.VMEM((2 * K,), jnp.int32),
        pltpu.VMEM((K, SROW), jnp.bfloat16),
        pltpu.VMEM((K, 2 * NH), jnp.float32),
        pltpu.VMEM((K, HID + 2 * NH), jnp.float32),
        pltpu.VMEM((K,), jnp.int32),
        pltpu.VMEM_SHARED((NPAD, HID + 2 * NH), jnp.float32),
        pltpu.SemaphoreType.DMA,
        pltpu.SemaphoreType.DMA,
        pltpu.SemaphoreType.DMA,
        pltpu.SemaphoreType.DMA,
    ],
)(_prop_body)


def _finish_body(acc_ref, r_ref, out_ref):
    s = acc_ref[0] + acc_ref[1]
    num = s[:, 0:HID]
    den = jnp.maximum(s[:, HID:HID + NH], 1e-16)
    dexp = jnp.dot(den, r_ref[...], preferred_element_type=jnp.float32)
    out_ref[...] = jnp.maximum(num / dexp, 0.0)


def _finish(ACC, RMAT):
    return pl.pallas_call(
        _finish_body,
        grid=(pl.cdiv(NPAD, FBR),),
        in_specs=[
            pl.BlockSpec((2, FBR, HID + 2 * NH), lambda i: (0, i, 0)),
            pl.BlockSpec((NH, HID), lambda i: (0, 0)),
        ],
        out_specs=pl.BlockSpec((FBR, HID), lambda i: (i, 0)),
        out_shape=jax.ShapeDtypeStruct((NPAD, HID), jnp.float32),
    )(ACC, RMAT)


def _pad_tab(t):
    return jnp.pad(t, ((0, NTAB - N), (0, 0)))


def kernel(x_author, x_paper, params, edge_writes, edge_rev):
    E = edge_writes.shape[1]
    epw = ((E + NW * 2 * K - 1) // (NW * 2 * K)) * 2 * K
    epad = NW * epw - E
    nch = epw // K
    fill = jnp.full((epad,), N, jnp.int32)

    def _ids(ei):
        s = jnp.concatenate([ei[0], fill]).reshape(NW, nch, K)
        d = jnp.concatenate([ei[1], fill]).reshape(NW, nch, K)
        return jnp.stack([s, d], axis=2).reshape(-1)

    ids1 = _ids(edge_writes)
    ids2 = _ids(edge_rev)

    smat = jnp.asarray(_S_NP)
    rmat = jnp.asarray(_S_NP.T)
    zeros = jnp.zeros((ROWS_PER_TILE, HID + 2 * NH), jnp.float32)

    xa, xp = x_author, x_paper
    for li in (1, 2):
        L = params['L%d' % li]
        X = jnp.concatenate([xa, xp], axis=0)
        W = jnp.stack([L['Wp_a'], L['Wp_p']])
        B = jnp.stack([L['bp_a'], L['bp_p']]).reshape(2, 1, HID)
        ASRC = jnp.stack([L['as_e1'].reshape(-1),
                          L['as_e2'].reshape(-1)]).reshape(2, 1, HID)
        ADST = jnp.stack([L['ad_e2'].reshape(-1),
                          L['ad_e1'].reshape(-1)]).reshape(2, 1, HID)
        H_, SS, SD, M = _prep(X, W, B, ASRC, ADST, smat)
        ST2, DT = _tables(SS, SD, M)
        feat = (H_.astype(jnp.bfloat16)
                .reshape(-1, 4, 2, HD).transpose(0, 1, 3, 2).reshape(-1, HID))
        shi = ST2.astype(jnp.bfloat16)
        slo = (ST2 - shi.astype(jnp.float32)).astype(jnp.bfloat16)
        sint = jnp.stack([shi, slo], axis=-1).reshape(-1, 2 * HD)
        ST = jnp.concatenate([feat, sint], axis=1)
        st_a, st_p = _pad_tab(ST[:N]), _pad_tab(ST[N:])
        dt_a, dt_p = _pad_tab(DT[:N]), _pad_tab(DT[N:])
        acc_p = _prop_sc(st_a, dt_p, ids1, zeros)
        acc_a = _prop_sc(st_p, dt_a, ids2, zeros)
        xa = _finish(acc_a, rmat)[:N]
        xp = _finish(acc_p, rmat)[:N]
    return xa, xp


# speedup vs baseline: 1.1589x; 1.0537x over previous
import functools

import jax
import jax.numpy as jnp
import numpy as np
from jax import lax
from jax.experimental import pallas as pl
from jax.experimental.pallas import tpu as pltpu
from jax.experimental.pallas import tpu_sc as plsc

N = 10000
HID = 128
NH = 8
HD = 16
NTAB = N + 8
NPAD = 10016
ROWS_PER_TILE = NPAD // 16
NW = 32
K = 80
SROW = 160
BR = 400
TPB = 25
FBR = 512

_S_NP = np.zeros((HID, NH), np.float32)
for _h in range(NH):
    _S_NP[_h * HD:(_h + 1) * HD, _h] = 1.0


def _leaky(x):
    return jnp.where(x < 0.0, x * 0.2, x)


def _prep_body(x_ref, w_ref, b_ref, asrc_ref, adst_ref, s_ref,
               h_ref, ss_ref, sd_ref, m_ref):
    i = pl.program_id(0)
    h = jnp.dot(x_ref[...], w_ref[0], preferred_element_type=jnp.float32)
    h = h + b_ref[0]
    h_ref[...] = h
    smat = s_ref[...]
    ss = jnp.dot(h * asrc_ref[0], smat, preferred_element_type=jnp.float32)
    sd = jnp.dot(h * adst_ref[0], smat, preferred_element_type=jnp.float32)
    ss_ref[...] = ss
    sd_ref[...] = sd
    cur = jnp.max(ss, axis=0, keepdims=True)

    @pl.when(i % TPB == 0)
    def _():
        m_ref[0] = cur

    @pl.when(i % TPB != 0)
    def _():
        m_ref[0] = jnp.maximum(m_ref[0], cur)


def _prep(X, W, B, ASRC, ADST, SMAT):
    return pl.pallas_call(
        _prep_body,
        grid=(2 * TPB,),
        in_specs=[
            pl.BlockSpec((BR, HID), lambda i: (i, 0)),
            pl.BlockSpec((1, HID, HID), lambda i: (i // TPB, 0, 0)),
            pl.BlockSpec((1, 1, HID), lambda i: (i // TPB, 0, 0)),
            pl.BlockSpec((1, 1, HID), lambda i: (i // TPB, 0, 0)),
            pl.BlockSpec((1, 1, HID), lambda i: (i // TPB, 0, 0)),
            pl.BlockSpec((HID, NH), lambda i: (0, 0)),
        ],
        out_specs=[
            pl.BlockSpec((BR, HID), lambda i: (i, 0)),
            pl.BlockSpec((BR, NH), lambda i: (i, 0)),
            pl.BlockSpec((BR, NH), lambda i: (i, 0)),
            pl.BlockSpec((1, 1, NH), lambda i: (i // TPB, 0, 0)),
        ],
        out_shape=[
            jax.ShapeDtypeStruct((2 * N, HID), jnp.float32),
            jax.ShapeDtypeStruct((2 * N, NH), jnp.float32),
            jax.ShapeDtypeStruct((2 * N, NH), jnp.float32),
            jax.ShapeDtypeStruct((2, 1, NH), jnp.float32),
        ],
    )(X, W, B, ASRC, ADST, SMAT)


def _tables_body(ss_ref, sd_ref, m_ref, st2_ref, dt_ref):
    ss = ss_ref[...]
    sd = sd_ref[...]
    st2_ref[...] = jnp.concatenate([ss, ss], axis=1)
    c = _leaky(m_ref[0] + sd)
    dt_ref[...] = jnp.concatenate([sd, -c], axis=1)


def _tables(SS, SD, M):
    return pl.pallas_call(
        _tables_body,
        grid=(2 * TPB,),
        in_specs=[
            pl.BlockSpec((BR, NH), lambda i: (i, 0)),
            pl.BlockSpec((BR, NH), lambda i: (i, 0)),
            pl.BlockSpec((1, 1, NH), lambda i: (1 - i // TPB, 0, 0)),
        ],
        out_specs=[
            pl.BlockSpec((BR, 2 * NH), lambda i: (i, 0)),
            pl.BlockSpec((BR, 2 * NH), lambda i: (i, 0)),
        ],
        out_shape=[
            jax.ShapeDtypeStruct((2 * N, 2 * NH), jnp.float32),
            jax.ShapeDtypeStruct((2 * N, 2 * NH), jnp.float32),
        ],
    )(SS, SD, M)


def _prop_body(src_tab, dst_tab, ids, zeros, out,
               iidx0, rows0, drows0, obuf0, didx0b,
               iidx1, rows1, drows1, obuf1, didx1b,
               acc, gsem0, gsem1, ssem0, ssem1, isem0, isem1):
    cid = lax.axis_index("c")
    sid = lax.axis_index("s")
    wid = sid * 2 + cid
    nch = ids.shape[0] // (NW * 2 * K)

    pltpu.sync_copy(zeros, acc.at[pl.ds(sid * ROWS_PER_TILE, ROWS_PER_TILE)])
    plsc.subcore_barrier()

    base0 = wid * nch * 2 * K
    bufs = ((iidx0, rows0, drows0, obuf0, didx0b, gsem0, ssem0, isem0),
            (iidx1, rows1, drows1, obuf1, didx1b, gsem1, ssem1, isem1))

    def fetch_idx(ci, b):
        i_, r_, dr_, ob_, d2_, gs, _, us = bufs[b]
        base = base0 + ci * 2 * K
        pltpu.async_copy(ids.at[pl.ds(base, 2 * K)], i_, us)

    def wait_idx(ci, b):
        i_, r_, dr_, ob_, d2_, gs, _, us = bufs[b]
        base = base0 + ci * 2 * K
        pltpu.make_async_copy(ids.at[pl.ds(base, 2 * K)], i_, us).wait()

    def issue_gather(b):
        i_, r_, dr_, ob_, d2_, gs, _, us = bufs[b]
        pltpu.async_copy(src_tab.at[i_.at[pl.ds(0, K)]], r_, gs)
        pltpu.async_copy(dst_tab.at[i_.at[pl.ds(K, K)]], dr_, gs)

    def wait_gather(b):
        i_, r_, dr_, ob_, d2_, gs, _, us = bufs[b]
        pltpu.make_async_copy(src_tab.at[i_.at[pl.ds(0, K)]], r_, gs).wait()
        pltpu.make_async_copy(dst_tab.at[i_.at[pl.ds(K, K)]], dr_, gs).wait()

    def wait_scatter(b):
        i_, r_, dr_, ob_, d2_, _, ss, us = bufs[b]
        pltpu.make_async_copy(ob_, acc.at[d2_], ss).wait()

    def compute(b):
        i_, r_, dr_, ob_, d2_, _, _, _ = bufs[b]
        rot_idx = (lax.iota(jnp.int32, HD) + NH) & (HD - 1)

        @plsc.parallel_loop(0, K, 1, unroll=4)
        def edge(e):
            shi, slo = plsc.unpack(r_[e, pl.ds(HID, 2 * HD)],
                                   format=plsc.PackFormat.INTERLEAVED,
                                   preferred_element_type=jnp.float32)
            ssv = shi + slo
            dv = dr_[e, pl.ds(0, HD)]
            rot = plsc.load_gather(dr_, [jnp.full((HD,), e, jnp.int32), rot_idx])
            t = ssv + dv
            t = jnp.where(t < 0.0, t * 0.2, t)
            w2 = jnp.exp(t + rot)
            ob_[e, pl.ds(HID, HD)] = w2
            for j in range(4):
                fv = r_[e, pl.ds(32 * j, 32)]
                va, vb = plsc.unpack(fv, format=plsc.PackFormat.INTERLEAVED,
                                     preferred_element_type=jnp.float32)
                ob_[e, pl.ds(32 * j, HD)] = va * w2[2 * j]
                ob_[e, pl.ds(32 * j + HD, HD)] = vb * w2[2 * j + 1]

    fetch_idx(0, 0)
    fetch_idx(1, 1)
    wait_idx(0, 0)
    issue_gather(0)

    def pair(pi, carry):
        for b in (0, 1):
            ci = pi * 2 + b
            i_, r_, dr_, ob_, d2_, _, ss, _us = bufs[b]

            @pl.when(ci + 1 < nch)
            def _():
                wait_idx(ci + 1, 1 - b)
                issue_gather(1 - b)

            wait_gather(b)

            @pl.when(ci >= 2)
            def _():
                wait_scatter(b)

            for q in range(K // HD):
                d2_[pl.ds(q * HD, HD)] = i_[pl.ds(K + q * HD, HD)]

            @pl.when(ci + 2 < nch)
            def _():
                fetch_idx(ci + 2, b)

            compute(b)
            pltpu.async_copy(ob_, acc.at[d2_], ss, add=True)
        return carry

    lax.fori_loop(0, nch // 2, pair, 0)
    wait_scatter(0)
    wait_scatter(1)
    plsc.subcore_barrier()
    pltpu.sync_copy(acc.at[pl.ds(sid * ROWS_PER_TILE, ROWS_PER_TILE)],
                    out.at[cid, pl.ds(sid * ROWS_PER_TILE, ROWS_PER_TILE)])


_prop_sc = functools.partial(
    pl.kernel,
    mesh=plsc.VectorSubcoreMesh(core_axis_name="c", subcore_axis_name="s"),
    compiler_params=pltpu.CompilerParams(use_tc_tiling_on_sc=False,
                                         needs_layout_passes=False),
    out_type=jax.ShapeDtypeStruct((2, NPAD, HID + 2 * NH), jnp.float32),
    scratch_types=[
        pltpu.VMEM((2 * K,), jnp.int32),
        pltpu.VMEM((K, SROW), jnp.bfloat16),
        pltpu.VMEM((K, 2 * NH), jnp.float32),
        pltpu.VMEM((K, HID + 2 * NH), jnp.float32),
        pltpu.VMEM((K,), jnp.int32),
        pltpu.VMEM((2 * K,), jnp.int32),
        pltpu.VMEM((K, SROW), jnp.bfloat16),
        pltpu.VMEM((K, 2 * NH), jnp.float32),
        pltpu.VMEM((K, HID + 2 * NH), jnp.float32),
        pltpu.VMEM((K,), jnp.int32),
        pltpu.VMEM_SHARED((NPAD, HID + 2 * NH), jnp.float32),
        pltpu.SemaphoreType.DMA,
        pltpu.SemaphoreType.DMA,
        pltpu.SemaphoreType.DMA,
        pltpu.SemaphoreType.DMA,
        pltpu.SemaphoreType.DMA,
        pltpu.SemaphoreType.DMA,
    ],
)(_prop_body)


def _finish_body(acc_ref, r_ref, out_ref):
    s = acc_ref[0] + acc_ref[1]
    num = s[:, 0:HID]
    den = jnp.maximum(s[:, HID:HID + NH], 1e-16)
    dexp = jnp.dot(den, r_ref[...], preferred_element_type=jnp.float32)
    out_ref[...] = jnp.maximum(num / dexp, 0.0)


def _finish(ACC, RMAT):
    return pl.pallas_call(
        _finish_body,
        grid=(pl.cdiv(NPAD, FBR),),
        in_specs=[
            pl.BlockSpec((2, FBR, HID + 2 * NH), lambda i: (0, i, 0)),
            pl.BlockSpec((NH, HID), lambda i: (0, 0)),
        ],
        out_specs=pl.BlockSpec((FBR, HID), lambda i: (i, 0)),
        out_shape=jax.ShapeDtypeStruct((NPAD, HID), jnp.float32),
    )(ACC, RMAT)


def _pad_tab(t):
    return jnp.pad(t, ((0, NTAB - N), (0, 0)))


def kernel(x_author, x_paper, params, edge_writes, edge_rev):
    E = edge_writes.shape[1]
    epw = ((E + NW * 2 * K - 1) // (NW * 2 * K)) * 2 * K
    epad = NW * epw - E
    nch = epw // K
    fill = jnp.full((epad,), N, jnp.int32)

    def _ids(ei):
        s = jnp.concatenate([ei[0], fill]).reshape(NW, nch, K)
        d = jnp.concatenate([ei[1], fill]).reshape(NW, nch, K)
        return jnp.stack([s, d], axis=2).reshape(-1)

    ids1 = _ids(edge_writes)
    ids2 = _ids(edge_rev)

    smat = jnp.asarray(_S_NP)
    rmat = jnp.asarray(_S_NP.T)
    zeros = jnp.zeros((ROWS_PER_TILE, HID + 2 * NH), jnp.float32)

    xa, xp = x_author, x_paper
    for li in (1, 2):
        L = params['L%d' % li]
        X = jnp.concatenate([xa, xp], axis=0)
        W = jnp.stack([L['Wp_a'], L['Wp_p']])
        B = jnp.stack([L['bp_a'], L['bp_p']]).reshape(2, 1, HID)
        ASRC = jnp.stack([L['as_e1'].reshape(-1),
                          L['as_e2'].reshape(-1)]).reshape(2, 1, HID)
        ADST = jnp.stack([L['ad_e2'].reshape(-1),
                          L['ad_e1'].reshape(-1)]).reshape(2, 1, HID)
        H_, SS, SD, M = _prep(X, W, B, ASRC, ADST, smat)
        ST2, DT = _tables(SS, SD, M)
        feat = (H_.astype(jnp.bfloat16)
                .reshape(-1, 4, 2, HD).transpose(0, 1, 3, 2).reshape(-1, HID))
        shi = ST2.astype(jnp.bfloat16)
        slo = (ST2 - shi.astype(jnp.float32)).astype(jnp.bfloat16)
        sint = jnp.stack([shi, slo], axis=-1).reshape(-1, 2 * HD)
        ST = jnp.concatenate([feat, sint], axis=1)
        st_a, st_p = _pad_tab(ST[:N]), _pad_tab(ST[N:])
        dt_a, dt_p = _pad_tab(DT[:N]), _pad_tab(DT[N:])
        acc_p = _prop_sc(st_a, dt_p, ids1, zeros)
        acc_a = _prop_sc(st_p, dt_a, ids2, zeros)
        xa = _finish(acc_a, rmat)[:N]
        xp = _finish(acc_p, rmat)[:N]
    return xa, xp


# speedup vs baseline: 1.1632x; 1.0038x over previous
import functools

import jax
import jax.numpy as jnp
import numpy as np
from jax import lax
from jax.experimental import pallas as pl
from jax.experimental.pallas import tpu as pltpu
from jax.experimental.pallas import tpu_sc as plsc

N = 10000
HID = 128
NH = 8
HD = 16
NTAB = N + 8
NPAD = 10016
ROWS_PER_TILE = NPAD // 16
NW = 32
K = 80
SROW = 160
BR = 400
TPB = 25
FBR = 512

_S_NP = np.zeros((HID, NH), np.float32)
for _h in range(NH):
    _S_NP[_h * HD:(_h + 1) * HD, _h] = 1.0


def _leaky(x):
    return jnp.where(x < 0.0, x * 0.2, x)


def _prep_body(x_ref, w_ref, b_ref, asrc_ref, adst_ref, s_ref,
               h_ref, ss_ref, sd_ref, m_ref):
    i = pl.program_id(0)
    h = jnp.dot(x_ref[...], w_ref[0], preferred_element_type=jnp.float32)
    h = h + b_ref[0]
    h_ref[...] = h
    smat = s_ref[...]
    ss = jnp.dot(h * asrc_ref[0], smat, preferred_element_type=jnp.float32)
    sd = jnp.dot(h * adst_ref[0], smat, preferred_element_type=jnp.float32)
    ss_ref[...] = ss
    sd_ref[...] = sd
    cur = jnp.max(ss, axis=0, keepdims=True)

    @pl.when(i % TPB == 0)
    def _():
        m_ref[0] = cur

    @pl.when(i % TPB != 0)
    def _():
        m_ref[0] = jnp.maximum(m_ref[0], cur)


def _prep(X, W, B, ASRC, ADST, SMAT):
    return pl.pallas_call(
        _prep_body,
        grid=(2 * TPB,),
        in_specs=[
            pl.BlockSpec((BR, HID), lambda i: (i, 0)),
            pl.BlockSpec((1, HID, HID), lambda i: (i // TPB, 0, 0)),
            pl.BlockSpec((1, 1, HID), lambda i: (i // TPB, 0, 0)),
            pl.BlockSpec((1, 1, HID), lambda i: (i // TPB, 0, 0)),
            pl.BlockSpec((1, 1, HID), lambda i: (i // TPB, 0, 0)),
            pl.BlockSpec((HID, NH), lambda i: (0, 0)),
        ],
        out_specs=[
            pl.BlockSpec((BR, HID), lambda i: (i, 0)),
            pl.BlockSpec((BR, NH), lambda i: (i, 0)),
            pl.BlockSpec((BR, NH), lambda i: (i, 0)),
            pl.BlockSpec((1, 1, NH), lambda i: (i // TPB, 0, 0)),
        ],
        out_shape=[
            jax.ShapeDtypeStruct((2 * N, HID), jnp.float32),
            jax.ShapeDtypeStruct((2 * N, NH), jnp.float32),
            jax.ShapeDtypeStruct((2 * N, NH), jnp.float32),
            jax.ShapeDtypeStruct((2, 1, NH), jnp.float32),
        ],
    )(X, W, B, ASRC, ADST, SMAT)


def _tables_body(ss_ref, sd_ref, m_ref, st2_ref, dt_ref):
    ss = ss_ref[...]
    sd = sd_ref[...]
    st2_ref[...] = jnp.concatenate([ss, ss], axis=1)
    c = _leaky(m_ref[0] + sd)
    dt_ref[...] = jnp.concatenate([sd, -c], axis=1)


def _tables(SS, SD, M):
    return pl.pallas_call(
        _tables_body,
        grid=(2 * TPB,),
        in_specs=[
            pl.BlockSpec((BR, NH), lambda i: (i, 0)),
            pl.BlockSpec((BR, NH), lambda i: (i, 0)),
            pl.BlockSpec((1, 1, NH), lambda i: (1 - i // TPB, 0, 0)),
        ],
        out_specs=[
            pl.BlockSpec((BR, 2 * NH), lambda i: (i, 0)),
            pl.BlockSpec((BR, 2 * NH), lambda i: (i, 0)),
        ],
        out_shape=[
            jax.ShapeDtypeStruct((2 * N, 2 * NH), jnp.float32),
            jax.ShapeDtypeStruct((2 * N, 2 * NH), jnp.float32),
        ],
    )(SS, SD, M)


def _prop_body(src_tab, dst_tab, ids, zeros, out,
               iidx0, rows0, drows0, obuf0, didx0b,
               iidx1, rows1, drows1, obuf1, didx1b,
               acc, gsem0, gsem1, ssem0, ssem1, isem0, isem1):
    cid = lax.axis_index("c")
    sid = lax.axis_index("s")
    wid = sid * 2 + cid
    nch = ids.shape[0] // (NW * 2 * K)

    pltpu.sync_copy(zeros, acc.at[pl.ds(sid * ROWS_PER_TILE, ROWS_PER_TILE)])
    plsc.subcore_barrier()

    base0 = wid * nch * 2 * K
    bufs = ((iidx0, rows0, drows0, obuf0, didx0b, gsem0, ssem0, isem0),
            (iidx1, rows1, drows1, obuf1, didx1b, gsem1, ssem1, isem1))

    def fetch_idx(ci, b):
        i_, r_, dr_, ob_, d2_, gs, _, us = bufs[b]
        base = base0 + ci * 2 * K
        pltpu.async_copy(ids.at[pl.ds(base, 2 * K)], i_, us)

    def wait_idx(ci, b):
        i_, r_, dr_, ob_, d2_, gs, _, us = bufs[b]
        base = base0 + ci * 2 * K
        pltpu.make_async_copy(ids.at[pl.ds(base, 2 * K)], i_, us).wait()

    def issue_gather(b):
        i_, r_, dr_, ob_, d2_, gs, _, us = bufs[b]
        pltpu.async_copy(src_tab.at[i_.at[pl.ds(0, K)]], r_, gs)
        pltpu.async_copy(dst_tab.at[i_.at[pl.ds(K, K)]], dr_, gs)

    def wait_gather(b):
        i_, r_, dr_, ob_, d2_, gs, _, us = bufs[b]
        pltpu.make_async_copy(src_tab.at[i_.at[pl.ds(0, K)]], r_, gs).wait()
        pltpu.make_async_copy(dst_tab.at[i_.at[pl.ds(K, K)]], dr_, gs).wait()

    def wait_scatter(b):
        i_, r_, dr_, ob_, d2_, _, ss, us = bufs[b]
        pltpu.make_async_copy(ob_, acc.at[d2_], ss).wait()

    def compute(b):
        i_, r_, dr_, ob_, d2_, _, _, _ = bufs[b]
        rot_idx = (lax.iota(jnp.int32, HD) + NH) & (HD - 1)

        @plsc.parallel_loop(0, K, 1, unroll=8)
        def edge(e):
            shi, slo = plsc.unpack(r_[e, pl.ds(HID, 2 * HD)],
                                   format=plsc.PackFormat.INTERLEAVED,
                                   preferred_element_type=jnp.float32)
            ssv = shi + slo
            dv = dr_[e, pl.ds(0, HD)]
            rot = plsc.load_gather(dr_, [jnp.full((HD,), e, jnp.int32), rot_idx])
            t = ssv + dv
            t = jnp.where(t < 0.0, t * 0.2, t)
            w2 = jnp.exp(t + rot)
            ob_[e, pl.ds(HID, HD)] = w2
            for j in range(4):
                fv = r_[e, pl.ds(32 * j, 32)]
                va, vb = plsc.unpack(fv, format=plsc.PackFormat.INTERLEAVED,
                                     preferred_element_type=jnp.float32)
                ob_[e, pl.ds(32 * j, HD)] = va * w2[2 * j]
                ob_[e, pl.ds(32 * j + HD, HD)] = vb * w2[2 * j + 1]

    fetch_idx(0, 0)
    fetch_idx(1, 1)
    wait_idx(0, 0)
    issue_gather(0)

    def pair(pi, carry):
        for b in (0, 1):
            ci = pi * 2 + b
            i_, r_, dr_, ob_, d2_, _, ss, _us = bufs[b]

            @pl.when(ci + 1 < nch)
            def _():
                wait_idx(ci + 1, 1 - b)
                issue_gather(1 - b)

            wait_gather(b)

            @pl.when(ci >= 2)
            def _():
                wait_scatter(b)

            for q in range(K // HD):
                d2_[pl.ds(q * HD, HD)] = i_[pl.ds(K + q * HD, HD)]

            @pl.when(ci + 2 < nch)
            def _():
                fetch_idx(ci + 2, b)

            compute(b)
            pltpu.async_copy(ob_, acc.at[d2_], ss, add=True)
        return carry

    lax.fori_loop(0, nch // 2, pair, 0)
    wait_scatter(0)
    wait_scatter(1)
    plsc.subcore_barrier()
    pltpu.sync_copy(acc.at[pl.ds(sid * ROWS_PER_TILE, ROWS_PER_TILE)],
                    out.at[cid, pl.ds(sid * ROWS_PER_TILE, ROWS_PER_TILE)])


_prop_sc = functools.partial(
    pl.kernel,
    mesh=plsc.VectorSubcoreMesh(core_axis_name="c", subcore_axis_name="s"),
    compiler_params=pltpu.CompilerParams(use_tc_tiling_on_sc=False,
                                         needs_layout_passes=False),
    out_type=jax.ShapeDtypeStruct((2, NPAD, HID + 2 * NH), jnp.float32),
    scratch_types=[
        pltpu.VMEM((2 * K,), jnp.int32),
        pltpu.VMEM((K, SROW), jnp.bfloat16),
        pltpu.VMEM((K, 2 * NH), jnp.float32),
        pltpu.VMEM((K, HID + 2 * NH), jnp.float32),
        pltpu.VMEM((K,), jnp.int32),
        pltpu.VMEM((2 * K,), jnp.int32),
        pltpu.VMEM((K, SROW), jnp.bfloat16),
        pltpu.VMEM((K, 2 * NH), jnp.float32),
        pltpu.VMEM((K, HID + 2 * NH), jnp.float32),
        pltpu.VMEM((K,), jnp.int32),
        pltpu.VMEM_SHARED((NPAD, HID + 2 * NH), jnp.float32),
        pltpu.SemaphoreType.DMA,
        pltpu.SemaphoreType.DMA,
        pltpu.SemaphoreType.DMA,
        pltpu.SemaphoreType.DMA,
        pltpu.SemaphoreType.DMA,
        pltpu.SemaphoreType.DMA,
    ],
)(_prop_body)


def _finish_body(acc_ref, r_ref, out_ref):
    s = acc_ref[0] + acc_ref[1]
    num = s[:, 0:HID]
    den = jnp.maximum(s[:, HID:HID + NH], 1e-16)
    dexp = jnp.dot(den, r_ref[...], preferred_element_type=jnp.float32)
    out_ref[...] = jnp.maximum(num / dexp, 0.0)


def _finish(ACC, RMAT):
    return pl.pallas_call(
        _finish_body,
        grid=(pl.cdiv(NPAD, FBR),),
        in_specs=[
            pl.BlockSpec((2, FBR, HID + 2 * NH), lambda i: (0, i, 0)),
            pl.BlockSpec((NH, HID), lambda i: (0, 0)),
        ],
        out_specs=pl.BlockSpec((FBR, HID), lambda i: (i, 0)),
        out_shape=jax.ShapeDtypeStruct((NPAD, HID), jnp.float32),
    )(ACC, RMAT)


def _pad_tab(t):
    return jnp.pad(t, ((0, NTAB - N), (0, 0)))


def kernel(x_author, x_paper, params, edge_writes, edge_rev):
    E = edge_writes.shape[1]
    epw = ((E + NW * 2 * K - 1) // (NW * 2 * K)) * 2 * K
    epad = NW * epw - E
    nch = epw // K
    fill = jnp.full((epad,), N, jnp.int32)

    def _ids(ei):
        s = jnp.concatenate([ei[0], fill]).reshape(NW, nch, K)
        d = jnp.concatenate([ei[1], fill]).reshape(NW, nch, K)
        return jnp.stack([s, d], axis=2).reshape(-1)

    ids1 = _ids(edge_writes)
    ids2 = _ids(edge_rev)

    smat = jnp.asarray(_S_NP)
    rmat = jnp.asarray(_S_NP.T)
    zeros = jnp.zeros((ROWS_PER_TILE, HID + 2 * NH), jnp.float32)

    xa, xp = x_author, x_paper
    for li in (1, 2):
        L = params['L%d' % li]
        X = jnp.concatenate([xa, xp], axis=0)
        W = jnp.stack([L['Wp_a'], L['Wp_p']])
        B = jnp.stack([L['bp_a'], L['bp_p']]).reshape(2, 1, HID)
        ASRC = jnp.stack([L['as_e1'].reshape(-1),
                          L['as_e2'].reshape(-1)]).reshape(2, 1, HID)
        ADST = jnp.stack([L['ad_e2'].reshape(-1),
                          L['ad_e1'].reshape(-1)]).reshape(2, 1, HID)
        H_, SS, SD, M = _prep(X, W, B, ASRC, ADST, smat)
        ST2, DT = _tables(SS, SD, M)
        feat = (H_.astype(jnp.bfloat16)
                .reshape(-1, 4, 2, HD).transpose(0, 1, 3, 2).reshape(-1, HID))
        shi = ST2.astype(jnp.bfloat16)
        slo = (ST2 - shi.astype(jnp.float32)).astype(jnp.bfloat16)
        sint = jnp.stack([shi, slo], axis=-1).reshape(-1, 2 * HD)
        ST = jnp.concatenate([feat, sint], axis=1)
        st_a, st_p = _pad_tab(ST[:N]), _pad_tab(ST[N:])
        dt_a, dt_p = _pad_tab(DT[:N]), _pad_tab(DT[N:])
        acc_p = _prop_sc(st_a, dt_p, ids1, zeros)
        acc_a = _prop_sc(st_p, dt_a, ids2, zeros)
        xa = _finish(acc_a, rmat)[:N]
        xp = _finish(acc_p, rmat)[:N]
    return xa, xp
